# baseline (speedup 1.0000x reference)
import dataclasses
import functools

import jax
import jax.numpy as jnp
from jax import lax
from jax.experimental import pallas as pl
from jax.experimental.pallas import tpu as pltpu
from jax.experimental.pallas import tpu_sc as plsc

LATENT = 128
NTYPES = 9
NC, NS = 2, 16
NW = NC * NS
CH = 128
TE = 4096
TN = 2048
EPS = 1e-5

def _sc_mesh():
    return plsc.VectorSubcoreMesh(core_axis_name="c", subcore_axis_name="s")


def _sc_params():
    cp = pltpu.CompilerParams()
    if "needs_layout_passes" in pltpu.CompilerParams.__dataclass_fields__:
        cp = dataclasses.replace(cp, needs_layout_passes=False)
    return cp


def _round_up(x, m):
    return (x + m - 1) // m * m


def sc_gather(table, idx):
    B = idx.shape[0]
    d = table.shape[1]
    dt = table.dtype
    per_w = B // NW
    n_ch = per_w // CH
    n_it = n_ch // 4

    @functools.partial(
        pl.kernel, mesh=_sc_mesh(),
        out_type=jax.ShapeDtypeStruct((B, d), dt),
        scratch_types=(
            [pltpu.VMEM((CH,), jnp.int32)] * 8
            + [pltpu.VMEM((CH, d), dt)] * 4
            + [pltpu.SemaphoreType.DMA] * 3
        ),
    )
    def k(table_hbm, idx_hbm, out_hbm, i0, i1, i2, i3, i4, i5, i6, i7,
          b0, b1, b2, b3, isem, gsem, osem):
        ib8 = (i0, i1, i2, i3, i4, i5, i6, i7)
        bb = (b0, b1, b2, b3)
        wid = lax.axis_index("s") * NC + lax.axis_index("c")
        base = wid * per_w

        def i_copy(c, iv):
            return pltpu.make_async_copy(
                idx_hbm.at[pl.ds(base + c * CH, CH)], iv, isem)

        def g_copy(iv, buf):
            return pltpu.make_async_copy(table_hbm.at[iv], buf, gsem)

        def s_copy(c, buf):
            return pltpu.make_async_copy(
                buf, out_hbm.at[pl.ds(base + c * CH, CH)], osem)

        def phase(vbase, r):
            c = vbase + r
            a0, a1 = bb[0], bb[1]
            g0, g1 = bb[2], bb[3]

            def slot(k):
                return ib8[(r + k) % 8]

            def ist(k):
                @pl.when(c + k < n_ch)
                def _():
                    i_copy(c + k, slot(k)).start()

            i_copy(c + 2, slot(2)).wait()
            i_copy(c + 3, slot(3)).wait()
            g_copy(slot(0), a0).wait()
            g_copy(slot(1), a1).wait()
            g_copy(slot(2), g0).start()
            g_copy(slot(3), g1).start()
            s_copy(c, a0).start()
            s_copy(c + 1, a1).start()
            ist(8)
            ist(9)
            s_copy(c, a0).wait()
            s_copy(c + 1, a1).wait()

            @pl.when(c + 4 < n_ch)
            def _():
                i_copy(c + 4, slot(4)).wait()
                i_copy(c + 5, slot(5)).wait()

            g_copy(slot(2), g0).wait()
            g_copy(slot(3), g1).wait()

            @pl.when(c + 4 < n_ch)
            def _():
                g_copy(slot(4), a0).start()
                g_copy(slot(5), a1).start()

            s_copy(c + 2, g0).start()
            s_copy(c + 3, g1).start()
            ist(10)
            ist(11)
            s_copy(c + 2, g0).wait()
            s_copy(c + 3, g1).wait()

        for kk in range(8):
            i_copy(kk, ib8[kk]).start()
        i_copy(0, ib8[0]).wait()
        i_copy(1, ib8[1]).wait()
        g_copy(ib8[0], bb[0]).start()
        g_copy(ib8[1], bb[1]).start()

        @pl.loop(0, n_ch // 8)
        def _(v):
            phase(8 * v, 0)
            phase(8 * v, 4)

    return k(table, idx)


def sc_segment_sum(edge_rows, ridx, npad):
    epad = edge_rows.shape[0]
    per_w = epad // NW
    n_ch = per_w // CH
    rows_per_tile = npad // NS
    n_out = rows_per_tile // CH

    @functools.partial(
        pl.kernel, mesh=_sc_mesh(),
        out_type=jax.ShapeDtypeStruct((2 * npad, LATENT), jnp.float32),
        scratch_types=[
            pltpu.VMEM((CH,), jnp.int32),
            pltpu.VMEM((CH,), jnp.int32),
            pltpu.VMEM((CH, LATENT), jnp.float32),
            pltpu.VMEM((CH, LATENT), jnp.float32),
            pltpu.VMEM_SHARED((npad, LATENT), jnp.float32),
            pltpu.SemaphoreType.DMA,
            pltpu.SemaphoreType.DMA,
        ],
    )
    def k(e_hbm, ridx_hbm, out_hbm, i0, i1, b0, b1, acc_sh, isem, lsem):
        cid = lax.axis_index("c")
        sid = lax.axis_index("s")
        base = cid * (epad // NC) + sid * per_w

        def i_copy(c, iv):
            return pltpu.make_async_copy(
                ridx_hbm.at[pl.ds(base + c * CH, CH)], iv, isem)

        def l_copy(c, buf):
            return pltpu.make_async_copy(
                e_hbm.at[pl.ds(base + c * CH, CH)], buf, lsem)

        @pl.loop(0, CH)
        def _(i):
            @pl.loop(0, LATENT // 16)
            def _(j):
                b0[i, pl.ds(j * 16, 16)] = jnp.zeros((16,), jnp.float32)

        for z in range(n_out):
            pltpu.sync_copy(
                b0, acc_sh.at[pl.ds(sid * rows_per_tile + z * CH, CH)])

        i_copy(0, i0).start()
        i_copy(1, i1).start()
        l_copy(0, b0).start()
        l_copy(1, b1).start()
        plsc.subcore_barrier()

        @pl.loop(0, n_ch // 2)
        def _(t):
            c = 2 * t
            i_copy(c, i0).wait()
            l_copy(c, b0).wait()
            pltpu.sync_copy(b0, acc_sh.at[i0], add=True)

            @pl.when(t + 1 < n_ch // 2)
            def _():
                i_copy(c + 2, i0).start()
                l_copy(c + 2, b0).start()

            i_copy(c + 1, i1).wait()
            l_copy(c + 1, b1).wait()
            pltpu.sync_copy(b1, acc_sh.at[i1], add=True)

            @pl.when(t + 1 < n_ch // 2)
            def _():
                i_copy(c + 3, i1).start()
                l_copy(c + 3, b1).start()

        plsc.subcore_barrier()

        for z in range(n_out):
            r0 = sid * rows_per_tile + z * CH
            pltpu.sync_copy(acc_sh.at[pl.ds(r0, CH)], b0)
            pltpu.sync_copy(b0, out_hbm.at[pl.ds(cid * npad + r0, CH)])

    return k(edge_rows, ridx)


def _dot(a, b):
    return jnp.dot(a, b, preferred_element_type=jnp.float32)


def _mlp_tail(h1, w2, b2, w3, b3, g, bb):
    h = jnp.maximum(h1, 0.0)
    h = jnp.maximum(_dot(h, w2) + b2, 0.0)
    h = _dot(h, w3) + b3
    mu = jnp.mean(h, axis=-1, keepdims=True)
    d = h - mu
    var = jnp.mean(d * d, axis=-1, keepdims=True)
    return d * lax.rsqrt(var + EPS) * g + bb


def _rep(shape):
    return pl.BlockSpec(shape, lambda i: tuple(0 for _ in shape))


def tc_encode_node(feat, w1, b1, w2, b2, w3, b3, g, bb, npad):
    def body(f_ref, w1r, b1r, w2r, b2r, w3r, b3r, gr, bbr, o_ref):
        h1 = _dot(f_ref[...], w1r[...]) + b1r[...]
        o_ref[...] = _mlp_tail(h1, w2r[...], b2r[...], w3r[...], b3r[...],
                               gr[...], bbr[...])

    return pl.pallas_call(
        body,
        grid=(npad // TN,),
        in_specs=[pl.BlockSpec((TN, 16), lambda i: (i, 0)),
                  _rep((16, LATENT)), _rep((1, LATENT)),
                  _rep((LATENT, LATENT)), _rep((1, LATENT)),
                  _rep((LATENT, LATENT)), _rep((1, LATENT)),
                  _rep((1, LATENT)), _rep((1, LATENT))],
        out_specs=pl.BlockSpec((TN, LATENT), lambda i: (i, 0)),
        out_shape=jax.ShapeDtypeStruct((npad, LATENT), jnp.float32),
    )(feat, w1, b1, w2, b2, w3, b3, g, bb)


def tc_encode_edge(tg, w1d, wnw, wnm, b1, w2, b2, w3, b3, g, bb, epad):
    nblk = epad // TE

    def body(ts_ref, tr_ref, w1r, wnwr, wnmr, b1r, w2r, b2r, w3r, b3r,
             gr, bbr, o_ref):
        d = ts_ref[...] - tr_ref[...]
        dsq = d * d
        lane = lax.broadcasted_iota(jnp.int32, (TE, LATENT), 1)
        mw = (lane < 3).astype(jnp.float32)
        mm = ((lane >= 4) & (lane < 6)).astype(jnp.float32)
        nw = jnp.sqrt(jnp.sum(dsq * mw, axis=-1, keepdims=True))
        nm = jnp.sqrt(jnp.sum(dsq * mm, axis=-1, keepdims=True))
        h1 = _dot(d, w1r[...]) + nw * wnwr[...] + nm * wnmr[...] + b1r[...]
        o_ref[...] = _mlp_tail(h1, w2r[...], b2r[...], w3r[...], b3r[...],
                               gr[...], bbr[...])

    return pl.pallas_call(
        body,
        grid=(nblk,),
        in_specs=[pl.BlockSpec((TE, LATENT), lambda i: (i, 0)),
                  pl.BlockSpec((TE, LATENT), lambda i, n=nblk: (i + n, 0)),
                  _rep((LATENT, LATENT)), _rep((1, LATENT)), _rep((1, LATENT)),
                  _rep((1, LATENT)),
                  _rep((LATENT, LATENT)), _rep((1, LATENT)),
                  _rep((LATENT, LATENT)), _rep((1, LATENT)),
                  _rep((1, LATENT)), _rep((1, LATENT))],
        out_specs=pl.BlockSpec((TE, LATENT), lambda i: (i, 0)),
        out_shape=jax.ShapeDtypeStruct((epad, LATENT), jnp.float32),
    )(tg, tg, w1d, wnw, wnm, b1, w2, b2, w3, b3, g, bb)


def tc_project(nl, ab, npad):
    def body(nl_ref, ab_ref, o_ref):
        o_ref[...] = _dot(nl_ref[...], ab_ref[0])[None]

    return pl.pallas_call(
        body,
        grid=(2, npad // TN),
        in_specs=[pl.BlockSpec((TN, LATENT), lambda j, i: (i, 0)),
                  pl.BlockSpec((1, LATENT, LATENT), lambda j, i: (j, 0, 0))],
        out_specs=pl.BlockSpec((1, TN, LATENT), lambda j, i: (j, i, 0)),
        out_shape=jax.ShapeDtypeStruct((2, npad, LATENT), jnp.float32),
    )(nl, ab)


def tc_edge_block(gath, elat, c, b1, w2, b2, w3, b3, g, bb, epad):
    nblk = epad // TE

    def body(gs_ref, gr_ref, e_ref, cr, b1r, w2r, b2r, w3r, b3r, gr_, bbr,
             o_ref):
        e = e_ref[...]
        h1 = gs_ref[...] + gr_ref[...] + _dot(e, cr[...]) + b1r[...]
        o_ref[...] = e + _mlp_tail(h1, w2r[...], b2r[...], w3r[...], b3r[...],
                                   gr_[...], bbr[...])

    return pl.pallas_call(
        body,
        grid=(nblk,),
        in_specs=[pl.BlockSpec((TE, LATENT), lambda i: (i, 0)),
                  pl.BlockSpec((TE, LATENT), lambda i, n=nblk: (i + n, 0)),
                  pl.BlockSpec((TE, LATENT), lambda i: (i, 0)),
                  _rep((LATENT, LATENT)), _rep((1, LATENT)),
                  _rep((LATENT, LATENT)), _rep((1, LATENT)),
                  _rep((LATENT, LATENT)), _rep((1, LATENT)),
                  _rep((1, LATENT)), _rep((1, LATENT))],
        out_specs=pl.BlockSpec((TE, LATENT), lambda i: (i, 0)),
        out_shape=jax.ShapeDtypeStruct((epad, LATENT), jnp.float32),
    )(gath, gath, elat, c, b1, w2, b2, w3, b3, g, bb)


def tc_node_block(nl, parts, d1, e1, b1, w2, b2, w3, b3, g, bb, npad):
    nblk = npad // TN

    def body(nl_ref, a0_ref, a1_ref, dr, er, b1r, w2r, b2r, w3r, b3r,
             gr, bbr, o_ref):
        nl_ = nl_ref[...]
        agg = a0_ref[...] + a1_ref[...]
        h1 = _dot(nl_, dr[...]) + _dot(agg, er[...]) + b1r[...]
        o_ref[...] = nl_ + _mlp_tail(h1, w2r[...], b2r[...], w3r[...],
                                     b3r[...], gr[...], bbr[...])

    return pl.pallas_call(
        body,
        grid=(nblk,),
        in_specs=[pl.BlockSpec((TN, LATENT), lambda i: (i, 0)),
                  pl.BlockSpec((TN, LATENT), lambda i: (i, 0)),
                  pl.BlockSpec((TN, LATENT), lambda i, n=nblk: (i + n, 0)),
                  _rep((LATENT, LATENT)), _rep((LATENT, LATENT)),
                  _rep((1, LATENT)),
                  _rep((LATENT, LATENT)), _rep((1, LATENT)),
                  _rep((LATENT, LATENT)), _rep((1, LATENT)),
                  _rep((1, LATENT)), _rep((1, LATENT))],
        out_specs=pl.BlockSpec((TN, LATENT), lambda i: (i, 0)),
        out_shape=jax.ShapeDtypeStruct((npad, LATENT), jnp.float32),
    )(nl, parts, parts, d1, e1, b1, w2, b2, w3, b3, g, bb)


def tc_decode(nl, w1, b1, w2, b2, w3p, b3p, npad):
    def body(nl_ref, w1r, b1r, w2r, b2r, w3r, b3r, o_ref):
        h = jnp.maximum(_dot(nl_ref[...], w1r[...]) + b1r[...], 0.0)
        h = jnp.maximum(_dot(h, w2r[...]) + b2r[...], 0.0)
        o_ref[...] = _dot(h, w3r[...]) + b3r[...]

    return pl.pallas_call(
        body,
        grid=(npad // TN,),
        in_specs=[pl.BlockSpec((TN, LATENT), lambda i: (i, 0)),
                  _rep((LATENT, LATENT)), _rep((1, LATENT)),
                  _rep((LATENT, LATENT)), _rep((1, LATENT)),
                  _rep((LATENT, LATENT)), _rep((1, LATENT))],
        out_specs=pl.BlockSpec((TN, LATENT), lambda i: (i, 0)),
        out_shape=jax.ShapeDtypeStruct((npad, LATENT), jnp.float32),
    )(nl, w1, b1, w2, b2, w3p, b3p)


def _row(v):
    return v.reshape(1, -1)


def kernel(world_pos, prev_world_pos, mesh_pos, node_type, edge_index, params):
    n = world_pos.shape[1]
    e = edge_index.shape[1]
    npad = _round_up(n, TN)
    epad = _round_up(e, max(TE, NW * CH))

    wp = world_pos[0]
    pwp = prev_world_pos[0]
    mp = mesh_pos[0]
    nt = node_type[0, :, 0]
    senders = edge_index[0]
    receivers = edge_index[1]

    pad_e = epad - e
    s_pad = jnp.concatenate(
        [senders, jnp.zeros((pad_e,), senders.dtype)]).astype(jnp.int32)
    r_pad = jnp.concatenate(
        [receivers, jnp.full((pad_e,), npad - 1, receivers.dtype)]
    ).astype(jnp.int32)
    ridx, s_sorted = lax.sort_key_val(r_pad, s_pad)
    idx_all = jnp.concatenate([s_sorted, ridx + npad])

    vel = wp - pwp
    onehot = jax.nn.one_hot(nt, NTYPES, dtype=jnp.float32)
    feat = jnp.concatenate([vel, onehot], axis=-1)
    feat = jnp.pad(feat, ((0, npad - n), (0, 16 - feat.shape[1])))

    p_enc_n = params['enc_node']
    n_std = params['node_norm_std']
    n_mean = params['node_norm_mean']
    w1n = p_enc_n['W1'] / n_std[:, None]
    b1n = _row(p_enc_n['b1'] - (n_mean / n_std) @ p_enc_n['W1'])
    w1n16 = jnp.zeros((16, LATENT), jnp.float32).at[:w1n.shape[0]].set(w1n)

    node_lat = tc_encode_node(
        feat, w1n16, b1n, p_enc_n['W2'], _row(p_enc_n['b2']),
        p_enc_n['W3'], _row(p_enc_n['b3']),
        _row(p_enc_n['ln_g']), _row(p_enc_n['ln_b']), npad)

    t_tab = jnp.zeros((npad, LATENT), jnp.float32)
    t_tab = t_tab.at[:n, 0:3].set(wp).at[:n, 4:6].set(mp)
    t2 = jnp.concatenate([t_tab, t_tab], axis=0)
    tg = sc_gather(t2, idx_all)

    p_enc_e = params['enc_edge']
    e_std = params['edge_norm_std']
    e_mean = params['edge_norm_mean']
    w1e = p_enc_e['W1'] / e_std[:, None]
    b1e = _row(p_enc_e['b1'] - (e_mean / e_std) @ p_enc_e['W1'])
    w1d = (jnp.zeros((LATENT, LATENT), jnp.float32)
           .at[0:3].set(w1e[0:3]).at[4:6].set(w1e[4:6]))

    edge_lat = tc_encode_edge(
        tg, w1d, _row(w1e[3]), _row(w1e[6]), b1e,
        p_enc_e['W2'], _row(p_enc_e['b2']), p_enc_e['W3'], _row(p_enc_e['b3']),
        _row(p_enc_e['ln_g']), _row(p_enc_e['ln_b']), epad)

    for blk in params['blocks']:
        pe = blk['edge']
        pn = blk['node']
        ab = jnp.stack([pe['W1'][:LATENT], pe['W1'][LATENT:2 * LATENT]])
        pq = tc_project(node_lat, ab, npad).reshape(2 * npad, LATENT)
        gath = sc_gather(pq, idx_all)
        edge_lat = tc_edge_block(
            gath, edge_lat, pe['W1'][2 * LATENT:], _row(pe['b1']),
            pe['W2'], _row(pe['b2']), pe['W3'], _row(pe['b3']),
            _row(pe['ln_g']), _row(pe['ln_b']), epad)
        parts = sc_segment_sum(edge_lat, ridx, npad)
        node_lat = tc_node_block(
            node_lat, parts, pn['W1'][:LATENT], pn['W1'][LATENT:],
            _row(pn['b1']), pn['W2'], _row(pn['b2']), pn['W3'], _row(pn['b3']),
            _row(pn['ln_g']), _row(pn['ln_b']), npad)

    p_dec = params['decoder']
    o_std = params['out_norm_std']
    o_mean = params['out_norm_mean']
    w3p = jnp.zeros((LATENT, LATENT), jnp.float32).at[:, :3].set(
        p_dec['W3'] * o_std[None, :])
    b3p = jnp.zeros((1, LATENT), jnp.float32).at[0, :3].set(
        p_dec['b3'] * o_std + o_mean)
    dec = tc_decode(node_lat, p_dec['W1'], _row(p_dec['b1']),
                    p_dec['W2'], _row(p_dec['b2']), w3p, b3p, npad)

    acc = dec[:n, 0:3][None]
    pred_pos = 2.0 * world_pos + acc - prev_world_pos
    return jnp.where(node_type == 0, pred_pos, world_pos)


# speedup vs baseline: 1.2214x; 1.2214x over previous
import functools

import jax
import jax.numpy as jnp
from jax import lax
from jax.experimental import pallas as pl
from jax.experimental.pallas import tpu as pltpu
from jax.experimental.pallas import tpu_sc as plsc

LATENT = 128
NTYPES = 9
NC, NS = 2, 16
NW = NC * NS
CH = 128
TE = 4096
TN = 2048
EPS = 1e-5

def _sc_mesh():
    return plsc.VectorSubcoreMesh(core_axis_name="c", subcore_axis_name="s")


def _round_up(x, m):
    return (x + m - 1) // m * m


def sc_gather(table, idx):
    B = idx.shape[0]
    d = table.shape[1]
    per_w = B // NW
    n_ch = per_w // CH
    n_it = n_ch // 4

    @functools.partial(
        pl.kernel, mesh=_sc_mesh(),
        out_type=jax.ShapeDtypeStruct((B, d), jnp.float32),
        scratch_types=[
            pltpu.VMEM((CH,), jnp.int32),
            pltpu.VMEM((CH,), jnp.int32),
            pltpu.VMEM((CH,), jnp.int32),
            pltpu.VMEM((CH,), jnp.int32),
            pltpu.VMEM((CH, d), jnp.float32),
            pltpu.VMEM((CH, d), jnp.float32),
            pltpu.VMEM((CH, d), jnp.float32),
            pltpu.VMEM((CH, d), jnp.float32),
            pltpu.SemaphoreType.DMA,
            pltpu.SemaphoreType.DMA,
            pltpu.SemaphoreType.DMA,
        ],
    )
    def k(table_hbm, idx_hbm, out_hbm, i0, i1, i2, i3,
          b0, b1, b2, b3, isem, gsem, osem):
        ib = (i0, i1, i2, i3)
        bb = (b0, b1, b2, b3)
        wid = lax.axis_index("s") * NC + lax.axis_index("c")
        base = wid * per_w

        def i_copy(c, iv):
            return pltpu.make_async_copy(
                idx_hbm.at[pl.ds(base + c * CH, CH)], iv, isem)

        def g_copy(iv, buf):
            return pltpu.make_async_copy(table_hbm.at[iv], buf, gsem)

        def s_copy(c, buf):
            return pltpu.make_async_copy(
                buf, out_hbm.at[pl.ds(base + c * CH, CH)], osem)

        for kk in range(4):
            i_copy(kk, ib[kk]).start()

        @pl.loop(0, n_it)
        def _(u):
            c = 4 * u
            for kk in range(4):
                i_copy(c + kk, ib[kk]).wait()
                g_copy(ib[kk], bb[kk]).start()
            for kk in range(4):
                g_copy(ib[kk], bb[kk]).wait()
                s_copy(c + kk, bb[kk]).start()

            @pl.when(u + 1 < n_it)
            def _():
                for kk in range(4):
                    i_copy(c + 4 + kk, ib[kk]).start()

            for kk in range(4):
                s_copy(c + kk, bb[kk]).wait()

    return k(table, idx)


def sc_segment_sum(edge_rows, ridx, npad):
    epad = edge_rows.shape[0]
    per_w = epad // NW
    n_ch = per_w // CH
    rows_per_tile = npad // NS
    n_out = rows_per_tile // CH

    @functools.partial(
        pl.kernel, mesh=_sc_mesh(),
        out_type=jax.ShapeDtypeStruct((2 * npad, LATENT), jnp.float32),
        scratch_types=[
            pltpu.VMEM((CH,), jnp.int32),
            pltpu.VMEM((CH,), jnp.int32),
            pltpu.VMEM((CH, LATENT), jnp.float32),
            pltpu.VMEM((CH, LATENT), jnp.float32),
            pltpu.VMEM_SHARED((npad, LATENT), jnp.float32),
            pltpu.SemaphoreType.DMA,
            pltpu.SemaphoreType.DMA,
        ],
    )
    def k(e_hbm, ridx_hbm, out_hbm, i0, i1, b0, b1, acc_sh, isem, lsem):
        cid = lax.axis_index("c")
        sid = lax.axis_index("s")
        base = cid * (epad // NC) + sid * per_w

        def i_copy(c, iv):
            return pltpu.make_async_copy(
                ridx_hbm.at[pl.ds(base + c * CH, CH)], iv, isem)

        def l_copy(c, buf):
            return pltpu.make_async_copy(
                e_hbm.at[pl.ds(base + c * CH, CH)], buf, lsem)

        @pl.loop(0, CH)
        def _(i):
            @pl.loop(0, LATENT // 16)
            def _(j):
                b0[i, pl.ds(j * 16, 16)] = jnp.zeros((16,), jnp.float32)

        for z in range(n_out):
            pltpu.sync_copy(
                b0, acc_sh.at[pl.ds(sid * rows_per_tile + z * CH, CH)])

        i_copy(0, i0).start()
        i_copy(1, i1).start()
        l_copy(0, b0).start()
        l_copy(1, b1).start()
        plsc.subcore_barrier()

        @pl.loop(0, n_ch // 2)
        def _(t):
            c = 2 * t
            i_copy(c, i0).wait()
            l_copy(c, b0).wait()
            pltpu.sync_copy(b0, acc_sh.at[i0], add=True)

            @pl.when(t + 1 < n_ch // 2)
            def _():
                i_copy(c + 2, i0).start()
                l_copy(c + 2, b0).start()

            i_copy(c + 1, i1).wait()
            l_copy(c + 1, b1).wait()
            pltpu.sync_copy(b1, acc_sh.at[i1], add=True)

            @pl.when(t + 1 < n_ch // 2)
            def _():
                i_copy(c + 3, i1).start()
                l_copy(c + 3, b1).start()

        plsc.subcore_barrier()

        for z in range(n_out):
            r0 = sid * rows_per_tile + z * CH
            pltpu.sync_copy(acc_sh.at[pl.ds(r0, CH)], b0)
            pltpu.sync_copy(b0, out_hbm.at[pl.ds(cid * npad + r0, CH)])

    return k(edge_rows, ridx)


def _dot(a, b):
    return jnp.dot(a, b, preferred_element_type=jnp.float32)


def _mlp_tail(h1, w2, b2, w3, b3, g, bb):
    h = jnp.maximum(h1, 0.0)
    h = jnp.maximum(_dot(h, w2) + b2, 0.0)
    h = _dot(h, w3) + b3
    mu = jnp.mean(h, axis=-1, keepdims=True)
    d = h - mu
    var = jnp.mean(d * d, axis=-1, keepdims=True)
    return d * lax.rsqrt(var + EPS) * g + bb


def _rep(shape):
    return pl.BlockSpec(shape, lambda i: tuple(0 for _ in shape))


def tc_encode_node(feat, w1, b1, w2, b2, w3, b3, g, bb, npad):
    def body(f_ref, w1r, b1r, w2r, b2r, w3r, b3r, gr, bbr, o_ref):
        h1 = _dot(f_ref[...], w1r[...]) + b1r[...]
        o_ref[...] = _mlp_tail(h1, w2r[...], b2r[...], w3r[...], b3r[...],
                               gr[...], bbr[...])

    return pl.pallas_call(
        body,
        grid=(npad // TN,),
        in_specs=[pl.BlockSpec((TN, 16), lambda i: (i, 0)),
                  _rep((16, LATENT)), _rep((1, LATENT)),
                  _rep((LATENT, LATENT)), _rep((1, LATENT)),
                  _rep((LATENT, LATENT)), _rep((1, LATENT)),
                  _rep((1, LATENT)), _rep((1, LATENT))],
        out_specs=pl.BlockSpec((TN, LATENT), lambda i: (i, 0)),
        out_shape=jax.ShapeDtypeStruct((npad, LATENT), jnp.float32),
    )(feat, w1, b1, w2, b2, w3, b3, g, bb)


def tc_encode_edge(tg, w1d, wnw, wnm, b1, w2, b2, w3, b3, g, bb, epad):
    nblk = epad // TE

    def body(ts_ref, tr_ref, w1r, wnwr, wnmr, b1r, w2r, b2r, w3r, b3r,
             gr, bbr, o_ref):
        d = ts_ref[...] - tr_ref[...]
        dsq = d * d
        lane = lax.broadcasted_iota(jnp.int32, (TE, LATENT), 1)
        mw = (lane < 3).astype(jnp.float32)
        mm = ((lane >= 4) & (lane < 6)).astype(jnp.float32)
        nw = jnp.sqrt(jnp.sum(dsq * mw, axis=-1, keepdims=True))
        nm = jnp.sqrt(jnp.sum(dsq * mm, axis=-1, keepdims=True))
        h1 = _dot(d, w1r[...]) + nw * wnwr[...] + nm * wnmr[...] + b1r[...]
        o_ref[...] = _mlp_tail(h1, w2r[...], b2r[...], w3r[...], b3r[...],
                               gr[...], bbr[...])

    return pl.pallas_call(
        body,
        grid=(nblk,),
        in_specs=[pl.BlockSpec((TE, LATENT), lambda i: (i, 0)),
                  pl.BlockSpec((TE, LATENT), lambda i, n=nblk: (i + n, 0)),
                  _rep((LATENT, LATENT)), _rep((1, LATENT)), _rep((1, LATENT)),
                  _rep((1, LATENT)),
                  _rep((LATENT, LATENT)), _rep((1, LATENT)),
                  _rep((LATENT, LATENT)), _rep((1, LATENT)),
                  _rep((1, LATENT)), _rep((1, LATENT))],
        out_specs=pl.BlockSpec((TE, LATENT), lambda i: (i, 0)),
        out_shape=jax.ShapeDtypeStruct((epad, LATENT), jnp.float32),
    )(tg, tg, w1d, wnw, wnm, b1, w2, b2, w3, b3, g, bb)


def tc_project(nl, ab, npad):
    def body(nl_ref, ab_ref, o_ref):
        o_ref[...] = _dot(nl_ref[...], ab_ref[0])[None]

    return pl.pallas_call(
        body,
        grid=(2, npad // TN),
        in_specs=[pl.BlockSpec((TN, LATENT), lambda j, i: (i, 0)),
                  pl.BlockSpec((1, LATENT, LATENT), lambda j, i: (j, 0, 0))],
        out_specs=pl.BlockSpec((1, TN, LATENT), lambda j, i: (j, i, 0)),
        out_shape=jax.ShapeDtypeStruct((2, npad, LATENT), jnp.float32),
    )(nl, ab)


def tc_edge_block(gath, elat, c, b1, w2, b2, w3, b3, g, bb, epad):
    nblk = epad // TE

    def body(gs_ref, gr_ref, e_ref, cr, b1r, w2r, b2r, w3r, b3r, gr_, bbr,
             o_ref):
        e = e_ref[...]
        h1 = gs_ref[...] + gr_ref[...] + _dot(e, cr[...]) + b1r[...]
        o_ref[...] = e + _mlp_tail(h1, w2r[...], b2r[...], w3r[...], b3r[...],
                                   gr_[...], bbr[...])

    return pl.pallas_call(
        body,
        grid=(nblk,),
        in_specs=[pl.BlockSpec((TE, LATENT), lambda i: (i, 0)),
                  pl.BlockSpec((TE, LATENT), lambda i, n=nblk: (i + n, 0)),
                  pl.BlockSpec((TE, LATENT), lambda i: (i, 0)),
                  _rep((LATENT, LATENT)), _rep((1, LATENT)),
                  _rep((LATENT, LATENT)), _rep((1, LATENT)),
                  _rep((LATENT, LATENT)), _rep((1, LATENT)),
                  _rep((1, LATENT)), _rep((1, LATENT))],
        out_specs=pl.BlockSpec((TE, LATENT), lambda i: (i, 0)),
        out_shape=jax.ShapeDtypeStruct((epad, LATENT), jnp.float32),
    )(gath, gath, elat, c, b1, w2, b2, w3, b3, g, bb)


def tc_node_block(nl, parts, d1, e1, b1, w2, b2, w3, b3, g, bb, ab_next,
                  npad):
    nblk = npad // TN

    def body(nl_ref, a0_ref, a1_ref, dr, er, b1r, w2r, b2r, w3r, b3r,
             gr, bbr, abn_ref, o_ref, pq_ref):
        nl_ = nl_ref[...]
        agg = a0_ref[...] + a1_ref[...]
        h1 = _dot(nl_, dr[...]) + _dot(agg, er[...]) + b1r[...]
        nl_new = nl_ + _mlp_tail(h1, w2r[...], b2r[...], w3r[...],
                                 b3r[...], gr[...], bbr[...])
        o_ref[...] = nl_new
        pq_ref[0] = _dot(nl_new, abn_ref[0])
        pq_ref[1] = _dot(nl_new, abn_ref[1])

    return pl.pallas_call(
        body,
        grid=(nblk,),
        in_specs=[pl.BlockSpec((TN, LATENT), lambda i: (i, 0)),
                  pl.BlockSpec((TN, LATENT), lambda i: (i, 0)),
                  pl.BlockSpec((TN, LATENT), lambda i, n=nblk: (i + n, 0)),
                  _rep((LATENT, LATENT)), _rep((LATENT, LATENT)),
                  _rep((1, LATENT)),
                  _rep((LATENT, LATENT)), _rep((1, LATENT)),
                  _rep((LATENT, LATENT)), _rep((1, LATENT)),
                  _rep((1, LATENT)), _rep((1, LATENT)),
                  _rep((2, LATENT, LATENT))],
        out_specs=[pl.BlockSpec((TN, LATENT), lambda i: (i, 0)),
                   pl.BlockSpec((2, TN, LATENT), lambda i: (0, i, 0))],
        out_shape=[jax.ShapeDtypeStruct((npad, LATENT), jnp.float32),
                   jax.ShapeDtypeStruct((2, npad, LATENT), jnp.float32)],
    )(nl, parts, parts, d1, e1, b1, w2, b2, w3, b3, g, bb, ab_next)


def tc_decode(nl, w1, b1, w2, b2, w3p, b3p, npad):
    def body(nl_ref, w1r, b1r, w2r, b2r, w3r, b3r, o_ref):
        h = jnp.maximum(_dot(nl_ref[...], w1r[...]) + b1r[...], 0.0)
        h = jnp.maximum(_dot(h, w2r[...]) + b2r[...], 0.0)
        o_ref[...] = _dot(h, w3r[...]) + b3r[...]

    return pl.pallas_call(
        body,
        grid=(npad // TN,),
        in_specs=[pl.BlockSpec((TN, LATENT), lambda i: (i, 0)),
                  _rep((LATENT, LATENT)), _rep((1, LATENT)),
                  _rep((LATENT, LATENT)), _rep((1, LATENT)),
                  _rep((LATENT, LATENT)), _rep((1, LATENT))],
        out_specs=pl.BlockSpec((TN, LATENT), lambda i: (i, 0)),
        out_shape=jax.ShapeDtypeStruct((npad, LATENT), jnp.float32),
    )(nl, w1, b1, w2, b2, w3p, b3p)


def _row(v):
    return v.reshape(1, -1)


def kernel(world_pos, prev_world_pos, mesh_pos, node_type, edge_index, params):
    n = world_pos.shape[1]
    e = edge_index.shape[1]
    npad = _round_up(n, TN)
    epad = _round_up(e, max(TE, NW * CH))

    wp = world_pos[0]
    pwp = prev_world_pos[0]
    mp = mesh_pos[0]
    nt = node_type[0, :, 0]
    senders = edge_index[0]
    receivers = edge_index[1]

    pad_e = epad - e
    s_pad = jnp.concatenate([senders, jnp.zeros((pad_e,), senders.dtype)])
    r_pad = jnp.concatenate([receivers, jnp.zeros((pad_e,), receivers.dtype)])
    idx_all = jnp.concatenate([s_pad, r_pad + npad]).astype(jnp.int32)
    ridx = jnp.concatenate(
        [receivers, jnp.full((pad_e,), npad - 1, receivers.dtype)]
    ).astype(jnp.int32)

    vel = wp - pwp
    onehot = jax.nn.one_hot(nt, NTYPES, dtype=jnp.float32)
    feat = jnp.concatenate([vel, onehot], axis=-1)
    feat = jnp.pad(feat, ((0, npad - n), (0, 16 - feat.shape[1])))

    p_enc_n = params['enc_node']
    n_std = params['node_norm_std']
    n_mean = params['node_norm_mean']
    w1n = p_enc_n['W1'] / n_std[:, None]
    b1n = _row(p_enc_n['b1'] - (n_mean / n_std) @ p_enc_n['W1'])
    w1n16 = jnp.zeros((16, LATENT), jnp.float32).at[:w1n.shape[0]].set(w1n)

    node_lat = tc_encode_node(
        feat, w1n16, b1n, p_enc_n['W2'], _row(p_enc_n['b2']),
        p_enc_n['W3'], _row(p_enc_n['b3']),
        _row(p_enc_n['ln_g']), _row(p_enc_n['ln_b']), npad)

    t_tab = jnp.zeros((npad, LATENT), jnp.float32)
    t_tab = t_tab.at[:n, 0:3].set(wp).at[:n, 4:6].set(mp)
    t2 = jnp.concatenate([t_tab, t_tab], axis=0)
    tg = sc_gather(t2, idx_all)

    p_enc_e = params['enc_edge']
    e_std = params['edge_norm_std']
    e_mean = params['edge_norm_mean']
    w1e = p_enc_e['W1'] / e_std[:, None]
    b1e = _row(p_enc_e['b1'] - (e_mean / e_std) @ p_enc_e['W1'])
    w1d = (jnp.zeros((LATENT, LATENT), jnp.float32)
           .at[0:3].set(w1e[0:3]).at[4:6].set(w1e[4:6]))

    edge_lat = tc_encode_edge(
        tg, w1d, _row(w1e[3]), _row(w1e[6]), b1e,
        p_enc_e['W2'], _row(p_enc_e['b2']), p_enc_e['W3'], _row(p_enc_e['b3']),
        _row(p_enc_e['ln_g']), _row(p_enc_e['ln_b']), epad)

    blocks = params['blocks']

    def _ab(pe):
        return jnp.stack([pe['W1'][:LATENT], pe['W1'][LATENT:2 * LATENT]])

    pq = tc_project(node_lat, _ab(blocks[0]['edge']), npad)
    for i, blk in enumerate(blocks):
        pe = blk['edge']
        pn = blk['node']
        gath = sc_gather(pq.reshape(2 * npad, LATENT), idx_all)
        edge_lat = tc_edge_block(
            gath, edge_lat, pe['W1'][2 * LATENT:], _row(pe['b1']),
            pe['W2'], _row(pe['b2']), pe['W3'], _row(pe['b3']),
            _row(pe['ln_g']), _row(pe['ln_b']), epad)
        parts = sc_segment_sum(edge_lat, ridx, npad)
        ab_next = _ab(blocks[i + 1]['edge'] if i + 1 < len(blocks) else pe)
        node_lat, pq = tc_node_block(
            node_lat, parts, pn['W1'][:LATENT], pn['W1'][LATENT:],
            _row(pn['b1']), pn['W2'], _row(pn['b2']), pn['W3'], _row(pn['b3']),
            _row(pn['ln_g']), _row(pn['ln_b']), ab_next, npad)

    p_dec = params['decoder']
    o_std = params['out_norm_std']
    o_mean = params['out_norm_mean']
    w3p = jnp.zeros((LATENT, LATENT), jnp.float32).at[:, :3].set(
        p_dec['W3'] * o_std[None, :])
    b3p = jnp.zeros((1, LATENT), jnp.float32).at[0, :3].set(
        p_dec['b3'] * o_std + o_mean)
    dec = tc_decode(node_lat, p_dec['W1'], _row(p_dec['b1']),
                    p_dec['W2'], _row(p_dec['b2']), w3p, b3p, npad)

    acc = dec[:n, 0:3][None]
    pred_pos = 2.0 * world_pos + acc - prev_world_pos
    return jnp.where(node_type == 0, pred_pos, world_pos)
